# baseline (speedup 1.0000x reference)
import functools

import jax
import jax.numpy as jnp
from jax import lax
from jax.experimental import pallas as pl
from jax.experimental.pallas import tpu as pltpu
from jax.experimental.pallas import tpu_sc as plsc

F32 = jnp.float32

N = 100000
N_PAD = 102400
E = 3200000
H = 16

NC = 2
NS = 16
NW = NC * NS

KB = 256
CH = 1024
JJ = CH // KB
EPW = E // NW
G_OUT = EPW // CH
T_KB = 32
T_CNT = (EPW - G_OUT * CH) // T_KB
ROWS_PER_TILE = N_PAD // NS


def _sc_mesh():
    return plsc.VectorSubcoreMesh(core_axis_name="c", subcore_axis_name="s",
                                  num_cores=NC)


_SC_PARAMS = pltpu.CompilerParams(use_tc_tiling_on_sc=False)


def _load_idx_rows(src_hbm, e0, buf, nrows, sem):
    for j in range(nrows):
        pltpu.async_copy(src_hbm.at[pl.ds(e0 + j * KB, KB)], buf.at[j], sem)


def _deg_body(dst1d, ew1d, zeros1d, deg_out, dstb, ewb, dstt, ewt, acc, lsem):
    c = lax.axis_index("c")
    s = lax.axis_index("s")
    wid = c * NS + s

    row0 = s * ROWS_PER_TILE
    pltpu.sync_copy(zeros1d.at[pl.ds(row0, ROWS_PER_TILE)],
                    acc.at[pl.ds(row0, ROWS_PER_TILE)])
    plsc.subcore_barrier()

    ebase = wid * EPW

    def load(g, b):
        e0 = ebase + g * CH
        _load_idx_rows(dst1d, e0, dstb.at[b], JJ, lsem)
        pltpu.async_copy(ew1d.at[pl.ds(e0, CH)], ewb.at[b], lsem)

    def wait_load(b):
        for j in range(JJ):
            pltpu.make_async_copy(dst1d.at[pl.ds(0, KB)], dstb.at[b, j],
                                  lsem).wait()
        pltpu.make_async_copy(ew1d.at[pl.ds(0, CH)], ewb.at[b], lsem).wait()

    load(0, 0)

    def outer(g, carry):
        b = lax.rem(g, 2)
        for bb in range(2):
            @pl.when(b == bb)
            def _():
                wait_load(bb)

                @pl.when(g + 1 < G_OUT)
                def _():
                    load(g + 1, 1 - bb)
                for j in range(JJ):
                    pltpu.sync_copy(ewb.at[bb, pl.ds(j * KB, KB)],
                                    acc.at[dstb.at[bb, j]], add=True)
        return carry

    lax.fori_loop(0, G_OUT, outer, 0)

    tbase = ebase + G_OUT * CH
    pltpu.sync_copy(ew1d.at[pl.ds(tbase, T_CNT * T_KB)], ewt)
    for t in range(T_CNT):
        pltpu.async_copy(dst1d.at[pl.ds(tbase + t * T_KB, T_KB)],
                         dstt.at[t], lsem)
    for t in range(T_CNT):
        pltpu.make_async_copy(dst1d.at[pl.ds(0, T_KB)], dstt.at[t],
                              lsem).wait()
    for t in range(T_CNT):
        pltpu.sync_copy(ewt.at[pl.ds(t * T_KB, T_KB)], acc.at[dstt.at[t]],
                        add=True)

    plsc.subcore_barrier()

    degv = _deg_body.degv
    rep = _deg_body.rep
    pltpu.sync_copy(acc.at[pl.ds(row0, ROWS_PER_TILE)], degv)

    def expand(k, carry):
        dv = degv[pl.ds(k * 16, 16)]
        for u in range(16):
            rep[k * 16 + u, :] = jnp.broadcast_to(dv[u], (H,))
        return carry

    lax.fori_loop(0, ROWS_PER_TILE // 16, expand, 0)
    pltpu.sync_copy(rep, deg_out.at[c, pl.ds(row0, ROWS_PER_TILE)])


def _run_deg(dst1d, ew1d, zeros1d):
    @functools.partial(
        pl.kernel,
        out_type=jax.ShapeDtypeStruct((NC, N_PAD, H), F32),
        mesh=_sc_mesh(),
        compiler_params=_SC_PARAMS,
        scratch_types=dict(
            dstb=pltpu.VMEM((2, JJ, KB), jnp.int32),
            ewb=pltpu.VMEM((2, CH), F32),
            dstt=pltpu.VMEM((T_CNT, T_KB), jnp.int32),
            ewt=pltpu.VMEM((T_CNT * T_KB,), F32),
            degv=pltpu.VMEM((ROWS_PER_TILE,), F32),
            rep=pltpu.VMEM((ROWS_PER_TILE, H), F32),
            acc=pltpu.VMEM_SHARED((N_PAD,), F32),
            lsem=pltpu.SemaphoreType.DMA,
        ),
    )
    def run(dst_h, ew_h, z_h, deg_h, *, dstb, ewb, dstt, ewt, degv, rep,
            acc, lsem):
        _deg_body.degv = degv
        _deg_body.rep = rep
        _deg_body(dst_h, ew_h, z_h, deg_h, dstb, ewb, dstt, ewt, acc, lsem)

    return run(dst1d, ew1d, zeros1d)


def _scale_rows(rows_ref, q, ewb_ref, b, j, nk):
    for k in range(nk):
        ew_v = ewb_ref[b, pl.ds(j * KB + k * 16, 16)]
        for u in range(16):
            rows_ref[q, k * 16 + u, :] = rows_ref[q, k * 16 + u, :] * ew_v[u]


def _agg_chunk_pipeline(hp, acc, srcb, dstb, ewb, rows, b, gsems, ssems):
    def gat(j, q):
        pltpu.async_copy(hp.at[srcb.at[b, pl.ds(j * KB, KB)]], rows.at[q],
                         gsems[q])

    def gat_wait(q):
        pltpu.make_async_copy(hp.at[srcb.at[b, pl.ds(0, KB)]], rows.at[q],
                              gsems[q]).wait()

    for j in range(JJ):
        gat(j, j)
    for j in range(JJ):
        gat_wait(j)
        _scale_rows(rows, j, ewb, b, j, KB // 16)
        pltpu.sync_copy(rows.at[j], acc.at[dstb.at[b, j]], add=True)


def _agg_body(hp, src1d, dst1d, ew1d, zeros16, acc_out,
              srcb, dstb, ewb, rows, srct, dstt, ewt, rowst, acc,
              lsem, gsems, ssems):
    c = lax.axis_index("c")
    s = lax.axis_index("s")
    wid = c * NS + s

    row0 = s * ROWS_PER_TILE
    pltpu.sync_copy(zeros16.at[pl.ds(row0, ROWS_PER_TILE)],
                    acc.at[pl.ds(row0, ROWS_PER_TILE)])
    plsc.subcore_barrier()

    ebase = wid * EPW

    def load(g, b):
        e0 = ebase + g * CH
        pltpu.async_copy(src1d.at[pl.ds(e0, CH)], srcb.at[b], lsem)
        _load_idx_rows(dst1d, e0, dstb.at[b], JJ, lsem)
        pltpu.async_copy(ew1d.at[pl.ds(e0, CH)], ewb.at[b], lsem)

    def wait_load(b):
        pltpu.make_async_copy(src1d.at[pl.ds(0, CH)], srcb.at[b], lsem).wait()
        for j in range(JJ):
            pltpu.make_async_copy(dst1d.at[pl.ds(0, KB)], dstb.at[b, j],
                                  lsem).wait()
        pltpu.make_async_copy(ew1d.at[pl.ds(0, CH)], ewb.at[b], lsem).wait()

    load(0, 0)

    def outer(g, carry):
        b = lax.rem(g, 2)
        for bb in range(2):
            @pl.when(b == bb)
            def _():
                wait_load(bb)

                @pl.when(g + 1 < G_OUT)
                def _():
                    load(g + 1, 1 - bb)
                _agg_chunk_pipeline(hp, acc, srcb, dstb, ewb, rows, bb,
                                    gsems, ssems)
        return carry

    lax.fori_loop(0, G_OUT, outer, 0)

    tbase = ebase + G_OUT * CH
    pltpu.sync_copy(ew1d.at[pl.ds(tbase, T_CNT * T_KB)], ewt)
    pltpu.sync_copy(src1d.at[pl.ds(tbase, T_CNT * T_KB)], srct)
    for t in range(T_CNT):
        pltpu.async_copy(dst1d.at[pl.ds(tbase + t * T_KB, T_KB)],
                         dstt.at[t], lsem)
    for t in range(T_CNT):
        pltpu.make_async_copy(dst1d.at[pl.ds(0, T_KB)], dstt.at[t],
                              lsem).wait()
    pltpu.async_copy(hp.at[srct.at[pl.ds(0, T_KB)]], rowst.at[0], gsems[0])
    for t in range(T_CNT):
        q = t % 2
        pltpu.make_async_copy(hp.at[srct.at[pl.ds(0, T_KB)]], rowst.at[q],
                              gsems[q]).wait()
        if t + 1 < T_CNT:
            pltpu.async_copy(hp.at[srct.at[pl.ds((t + 1) * T_KB, T_KB)]],
                             rowst.at[1 - q], gsems[1 - q])

        def tmul(k, _, q=q, t=t):
            ew_v = ewt[pl.ds(t * T_KB + k * 16, 16)]
            for u in range(16):
                rowst[q, k * 16 + u, :] = rowst[q, k * 16 + u, :] * ew_v[u]
            return 0

        lax.fori_loop(0, T_KB // 16, tmul, 0)
        pltpu.sync_copy(rowst.at[q], acc.at[dstt.at[t]], add=True)

    plsc.subcore_barrier()
    pltpu.sync_copy(acc.at[pl.ds(row0, ROWS_PER_TILE)],
                    acc_out.at[c, pl.ds(row0, ROWS_PER_TILE)])


def _run_agg(hp, src1d, dst1d, ew1d, zeros16):
    @functools.partial(
        pl.kernel,
        out_type=jax.ShapeDtypeStruct((NC, N_PAD, H), F32),
        mesh=_sc_mesh(),
        compiler_params=_SC_PARAMS,
        scratch_types=dict(
            srcb=pltpu.VMEM((2, CH), jnp.int32),
            dstb=pltpu.VMEM((2, JJ, KB), jnp.int32),
            ewb=pltpu.VMEM((2, CH), F32),
            rows=pltpu.VMEM((4, KB, H), F32),
            srct=pltpu.VMEM((T_CNT * T_KB,), jnp.int32),
            dstt=pltpu.VMEM((T_CNT, T_KB), jnp.int32),
            ewt=pltpu.VMEM((T_CNT * T_KB,), F32),
            rowst=pltpu.VMEM((2, T_KB, H), F32),
            acc=pltpu.VMEM_SHARED((N_PAD, H), F32),
            lsem=pltpu.SemaphoreType.DMA,
            gs0=pltpu.SemaphoreType.DMA,
            gs1=pltpu.SemaphoreType.DMA,
            gs2=pltpu.SemaphoreType.DMA,
            gs3=pltpu.SemaphoreType.DMA,
            ss0=pltpu.SemaphoreType.DMA,
            ss1=pltpu.SemaphoreType.DMA,
            ss2=pltpu.SemaphoreType.DMA,
            ss3=pltpu.SemaphoreType.DMA,
        ),
    )
    def run(hp_h, src_h, dst_h, ew_h, z_h, acc_h,
            *, srcb, dstb, ewb, rows, srct, dstt, ewt, rowst, acc,
            lsem, gs0, gs1, gs2, gs3, ss0, ss1, ss2, ss3):
        _agg_body(hp_h, src_h, dst_h, ew_h, z_h, acc_h,
                  srcb, dstb, ewb, rows, srct, dstt, ewt, rowst, acc,
                  lsem, [gs0, gs1, gs2, gs3], [ss0, ss1, ss2, ss3])

    return run(hp, src1d, dst1d, ew1d, zeros16)


FLAT = N_PAD * H // 128
_FB = 256
_GRID = FLAT // _FB


def _prologue_tc(deg_ref, x_ref, wbd_ref, dis_ref, hp_ref):
    deg = deg_ref[0] + deg_ref[1] + 1.0
    dis = lax.rsqrt(deg)
    dis_ref[...] = dis
    h = jnp.dot(x_ref[...], wbd_ref[...], preferred_element_type=F32)
    hp_ref[...] = h * dis


def _run_prologue(deg_f, x_f, wbd1):
    return pl.pallas_call(
        _prologue_tc,
        grid=(_GRID,),
        in_specs=[
            pl.BlockSpec((NC, _FB, 128), lambda i: (0, i, 0)),
            pl.BlockSpec((_FB, 128), lambda i: (i, 0)),
            pl.BlockSpec((128, 128), lambda i: (0, 0)),
        ],
        out_specs=[
            pl.BlockSpec((_FB, 128), lambda i: (i, 0)),
            pl.BlockSpec((_FB, 128), lambda i: (i, 0)),
        ],
        out_shape=[
            jax.ShapeDtypeStruct((FLAT, 128), F32),
            jax.ShapeDtypeStruct((FLAT, 128), F32),
        ],
    )(deg_f, x_f, wbd1)


def _layer_tc(acc_ref, hp_ref, dis_ref, b_ref, wbd_ref, out_ref, *, last):
    s = acc_ref[0] + acc_ref[1] + hp_ref[...]
    v = dis_ref[...] * s + b_ref[...]
    x = jnp.where(v >= 0, v, 0.01 * v)
    if last:
        out_ref[...] = x
    else:
        h = jnp.dot(x, wbd_ref[...], preferred_element_type=F32)
        out_ref[...] = h * dis_ref[...]


def _run_layer(acc_f, hp_f, dis_f, b_f, wbd_next, last):
    return pl.pallas_call(
        functools.partial(_layer_tc, last=last),
        grid=(_GRID,),
        in_specs=[
            pl.BlockSpec((NC, _FB, 128), lambda i: (0, i, 0)),
            pl.BlockSpec((_FB, 128), lambda i: (i, 0)),
            pl.BlockSpec((_FB, 128), lambda i: (i, 0)),
            pl.BlockSpec((1, 128), lambda i: (0, 0)),
            pl.BlockSpec((128, 128), lambda i: (0, 0)),
        ],
        out_specs=pl.BlockSpec((_FB, 128), lambda i: (i, 0)),
        out_shape=jax.ShapeDtypeStruct((FLAT, 128), F32),
    )(acc_f, hp_f, dis_f, b_f, wbd_next)


def _head_tc(xa_ref, xb_ref, w1_ref, b1_ref, w2_ref, b2_ref, w3_ref, b3_ref,
             za_ref, penal_ref):
    def mlp(xg):
        z = jnp.dot(xg, w1_ref[...], preferred_element_type=F32) + b1_ref[...]
        z = jnp.dot(z, w2_ref[...], preferred_element_type=F32) + b2_ref[...]
        z = jnp.dot(z, w3_ref[...], preferred_element_type=F32) + b3_ref[...]
        return z

    za = mlp(xa_ref[...])
    zb = mlp(xb_ref[...])
    za_ref[...] = za
    d = za - zb
    ss = jnp.sum(d * d)
    penal = 0.09 * xa_ref.shape[0] * lax.rsqrt(ss)
    penal_ref[...] = jnp.reshape(penal, (1, 1))


def _run_head(xa, xb, w1, b1, w2, b2, w3, b3):
    rows = xa.shape[0]
    return pl.pallas_call(
        _head_tc,
        out_shape=[
            jax.ShapeDtypeStruct((rows, 128), F32),
            jax.ShapeDtypeStruct((1, 1), F32),
        ],
    )(xa, xb, w1, b1, w2, b2, w3, b3)


def kernel(x, edge_index, edge_weight, W1, b1, W2, b2, W3, b3, W4, b4,
           fc1_w, fc1_b, fc2_w, fc2_b, fc3_w, fc3_b):
    src = edge_index[0]
    dst = edge_index[1]

    zeros1d = jnp.zeros((N_PAD,), F32)
    zeros16 = jnp.zeros((N_PAD, H), F32)
    eye8 = jnp.eye(8, dtype=F32)
    wbds = [jnp.kron(eye8, w) for w in (W1, W2, W3, W4)]
    bfs = [jnp.tile(b, 8).reshape(1, 128) for b in (b1, b2, b3, b4)]

    x_f = jnp.pad(x.reshape(N * H // 128, 128),
                  ((0, FLAT - N * H // 128), (0, 0)))

    deg16 = _run_deg(dst, edge_weight, zeros1d)
    deg_f = deg16.reshape(NC, FLAT, 128)
    dis_f, hpf = _run_prologue(deg_f, x_f, wbds[0])

    for l in range(4):
        accp = _run_agg(hpf.reshape(N_PAD, H), src, dst, edge_weight,
                        zeros16)
        hpf = _run_layer(accp.reshape(NC, FLAT, 128), hpf, dis_f, bfs[l],
                         wbds[min(l + 1, 3)], last=(l == 3))

    xa = hpf[:N * H // 128].reshape(-1, 1600)
    xb = x.reshape(-1, 1600)
    fc3_wp = jnp.pad(fc3_w, ((0, 0), (0, 126)))
    fc3_bp = jnp.pad(fc3_b, (0, 126))
    za, penal = _run_head(xa, xb,
                          fc1_w, fc1_b.reshape(1, -1),
                          fc2_w, fc2_b.reshape(1, -1),
                          fc3_wp, fc3_bp.reshape(1, -1))
    x_cls = za[:, :2]
    return (x_cls, penal[0, 0])


# speedup vs baseline: 1.0181x; 1.0181x over previous
import functools

import jax
import jax.numpy as jnp
from jax import lax
from jax.experimental import pallas as pl
from jax.experimental.pallas import tpu as pltpu
from jax.experimental.pallas import tpu_sc as plsc

F32 = jnp.float32

N = 100000
N_PAD = 102400
E = 3200000
H = 16

NC = 2
NS = 16
NW = NC * NS

KB = 256
CH = 1024
JJ = CH // KB
EPW = E // NW
G_OUT = EPW // CH
T_KB = 32
T_CNT = (EPW - G_OUT * CH) // T_KB
ROWS_PER_TILE = N_PAD // NS


def _sc_mesh():
    return plsc.VectorSubcoreMesh(core_axis_name="c", subcore_axis_name="s",
                                  num_cores=NC)


_SC_PARAMS = pltpu.CompilerParams(use_tc_tiling_on_sc=False)


def _load_idx_rows(src_hbm, e0, buf, nrows, sem):
    for j in range(nrows):
        pltpu.async_copy(src_hbm.at[pl.ds(e0 + j * KB, KB)], buf.at[j], sem)


def _deg_body(dst1d, ew1d, zeros1d, deg_out, dstb, ewb, dstt, ewt, acc, lsem):
    c = lax.axis_index("c")
    s = lax.axis_index("s")
    wid = c * NS + s

    row0 = s * ROWS_PER_TILE
    pltpu.sync_copy(zeros1d.at[pl.ds(row0, ROWS_PER_TILE)],
                    acc.at[pl.ds(row0, ROWS_PER_TILE)])
    plsc.subcore_barrier()

    ebase = wid * EPW

    def load(g, b):
        e0 = ebase + g * CH
        _load_idx_rows(dst1d, e0, dstb.at[b], JJ, lsem)
        pltpu.async_copy(ew1d.at[pl.ds(e0, CH)], ewb.at[b], lsem)

    def wait_load(b):
        for j in range(JJ):
            pltpu.make_async_copy(dst1d.at[pl.ds(0, KB)], dstb.at[b, j],
                                  lsem).wait()
        pltpu.make_async_copy(ew1d.at[pl.ds(0, CH)], ewb.at[b], lsem).wait()

    load(0, 0)

    def outer(g, carry):
        b = lax.rem(g, 2)
        for bb in range(2):
            @pl.when(b == bb)
            def _():
                wait_load(bb)

                @pl.when(g + 1 < G_OUT)
                def _():
                    load(g + 1, 1 - bb)
                for j in range(JJ):
                    pltpu.sync_copy(ewb.at[bb, pl.ds(j * KB, KB)],
                                    acc.at[dstb.at[bb, j]], add=True)
        return carry

    lax.fori_loop(0, G_OUT, outer, 0)

    tbase = ebase + G_OUT * CH
    pltpu.sync_copy(ew1d.at[pl.ds(tbase, T_CNT * T_KB)], ewt)
    for t in range(T_CNT):
        pltpu.async_copy(dst1d.at[pl.ds(tbase + t * T_KB, T_KB)],
                         dstt.at[t], lsem)
    for t in range(T_CNT):
        pltpu.make_async_copy(dst1d.at[pl.ds(0, T_KB)], dstt.at[t],
                              lsem).wait()
    for t in range(T_CNT):
        pltpu.sync_copy(ewt.at[pl.ds(t * T_KB, T_KB)], acc.at[dstt.at[t]],
                        add=True)

    plsc.subcore_barrier()

    degv = _deg_body.degv
    rep = _deg_body.rep
    pltpu.sync_copy(acc.at[pl.ds(row0, ROWS_PER_TILE)], degv)

    def expand(k, carry):
        dv = degv[pl.ds(k * 16, 16)]
        for u in range(16):
            rep[k * 16 + u, :] = jnp.broadcast_to(dv[u], (H,))
        return carry

    lax.fori_loop(0, ROWS_PER_TILE // 16, expand, 0)
    pltpu.sync_copy(rep, deg_out.at[c, pl.ds(row0, ROWS_PER_TILE)])


def _run_deg(dst1d, ew1d, zeros1d):
    @functools.partial(
        pl.kernel,
        out_type=jax.ShapeDtypeStruct((NC, N_PAD, H), F32),
        mesh=_sc_mesh(),
        compiler_params=_SC_PARAMS,
        scratch_types=dict(
            dstb=pltpu.VMEM((2, JJ, KB), jnp.int32),
            ewb=pltpu.VMEM((2, CH), F32),
            dstt=pltpu.VMEM((T_CNT, T_KB), jnp.int32),
            ewt=pltpu.VMEM((T_CNT * T_KB,), F32),
            degv=pltpu.VMEM((ROWS_PER_TILE,), F32),
            rep=pltpu.VMEM((ROWS_PER_TILE, H), F32),
            acc=pltpu.VMEM_SHARED((N_PAD,), F32),
            lsem=pltpu.SemaphoreType.DMA,
        ),
    )
    def run(dst_h, ew_h, z_h, deg_h, *, dstb, ewb, dstt, ewt, degv, rep,
            acc, lsem):
        _deg_body.degv = degv
        _deg_body.rep = rep
        _deg_body(dst_h, ew_h, z_h, deg_h, dstb, ewb, dstt, ewt, acc, lsem)

    return run(dst1d, ew1d, zeros1d)


def _scale_rows(rows_ref, q, ewb_ref, b, j, nk):
    for k in range(nk):
        ew_v = ewb_ref[b, pl.ds(j * KB + k * 16, 16)]
        for u in range(16):
            rows_ref[q, k * 16 + u, :] = rows_ref[q, k * 16 + u, :] * ew_v[u]


def _agg_chunk_pipeline(hp, acc, srcb, dstb, ewb, rows, b, gsems, ssems):
    def gat(j, q):
        pltpu.async_copy(hp.at[srcb.at[b, pl.ds(j * KB, KB)]], rows.at[q],
                         gsems[q])

    def gat_wait(q):
        pltpu.make_async_copy(hp.at[srcb.at[b, pl.ds(0, KB)]], rows.at[q],
                              gsems[q]).wait()

    gat(0, 0)
    gat(1, 1)
    gat(2, 2)
    for j in range(JJ):
        q = j % 4
        gat_wait(q)
        _scale_rows(rows, q, ewb, b, j, KB // 16)
        pltpu.sync_copy(rows.at[q], acc.at[dstb.at[b, j]], add=True)
        if j + 3 < JJ:
            gat(j + 3, (j + 3) % 4)


def _agg_body(hp, src1d, dst1d, ew1d, zeros16, acc_out,
              srcb, dstb, ewb, rows, srct, dstt, ewt, rowst, acc,
              lsem, gsems, ssems):
    c = lax.axis_index("c")
    s = lax.axis_index("s")
    wid = c * NS + s

    row0 = s * ROWS_PER_TILE
    pltpu.sync_copy(zeros16.at[pl.ds(row0, ROWS_PER_TILE)],
                    acc.at[pl.ds(row0, ROWS_PER_TILE)])
    plsc.subcore_barrier()

    ebase = wid * EPW

    def load(g, b):
        e0 = ebase + g * CH
        pltpu.async_copy(src1d.at[pl.ds(e0, CH)], srcb.at[b], lsem)
        _load_idx_rows(dst1d, e0, dstb.at[b], JJ, lsem)
        pltpu.async_copy(ew1d.at[pl.ds(e0, CH)], ewb.at[b], lsem)

    def wait_load(b):
        pltpu.make_async_copy(src1d.at[pl.ds(0, CH)], srcb.at[b], lsem).wait()
        for j in range(JJ):
            pltpu.make_async_copy(dst1d.at[pl.ds(0, KB)], dstb.at[b, j],
                                  lsem).wait()
        pltpu.make_async_copy(ew1d.at[pl.ds(0, CH)], ewb.at[b], lsem).wait()

    load(0, 0)

    def outer(g, carry):
        b = lax.rem(g, 2)
        for bb in range(2):
            @pl.when(b == bb)
            def _():
                wait_load(bb)

                @pl.when(g + 1 < G_OUT)
                def _():
                    load(g + 1, 1 - bb)
                _agg_chunk_pipeline(hp, acc, srcb, dstb, ewb, rows, bb,
                                    gsems, ssems)
        return carry

    lax.fori_loop(0, G_OUT, outer, 0)

    tbase = ebase + G_OUT * CH
    pltpu.sync_copy(ew1d.at[pl.ds(tbase, T_CNT * T_KB)], ewt)
    pltpu.sync_copy(src1d.at[pl.ds(tbase, T_CNT * T_KB)], srct)
    for t in range(T_CNT):
        pltpu.async_copy(dst1d.at[pl.ds(tbase + t * T_KB, T_KB)],
                         dstt.at[t], lsem)
    for t in range(T_CNT):
        pltpu.make_async_copy(dst1d.at[pl.ds(0, T_KB)], dstt.at[t],
                              lsem).wait()
    pltpu.async_copy(hp.at[srct.at[pl.ds(0, T_KB)]], rowst.at[0], gsems[0])
    for t in range(T_CNT):
        q = t % 2
        pltpu.make_async_copy(hp.at[srct.at[pl.ds(0, T_KB)]], rowst.at[q],
                              gsems[q]).wait()
        if t + 1 < T_CNT:
            pltpu.async_copy(hp.at[srct.at[pl.ds((t + 1) * T_KB, T_KB)]],
                             rowst.at[1 - q], gsems[1 - q])

        def tmul(k, _, q=q, t=t):
            ew_v = ewt[pl.ds(t * T_KB + k * 16, 16)]
            for u in range(16):
                rowst[q, k * 16 + u, :] = rowst[q, k * 16 + u, :] * ew_v[u]
            return 0

        lax.fori_loop(0, T_KB // 16, tmul, 0)
        pltpu.sync_copy(rowst.at[q], acc.at[dstt.at[t]], add=True)

    plsc.subcore_barrier()
    pltpu.sync_copy(acc.at[pl.ds(row0, ROWS_PER_TILE)],
                    acc_out.at[c, pl.ds(row0, ROWS_PER_TILE)])


def _run_agg(hp, src1d, dst1d, ew1d, zeros16):
    @functools.partial(
        pl.kernel,
        out_type=jax.ShapeDtypeStruct((NC, N_PAD, H), F32),
        mesh=_sc_mesh(),
        compiler_params=_SC_PARAMS,
        scratch_types=dict(
            srcb=pltpu.VMEM((2, CH), jnp.int32),
            dstb=pltpu.VMEM((2, JJ, KB), jnp.int32),
            ewb=pltpu.VMEM((2, CH), F32),
            rows=pltpu.VMEM((4, KB, H), F32),
            srct=pltpu.VMEM((T_CNT * T_KB,), jnp.int32),
            dstt=pltpu.VMEM((T_CNT, T_KB), jnp.int32),
            ewt=pltpu.VMEM((T_CNT * T_KB,), F32),
            rowst=pltpu.VMEM((2, T_KB, H), F32),
            acc=pltpu.VMEM_SHARED((N_PAD, H), F32),
            lsem=pltpu.SemaphoreType.DMA,
            gs0=pltpu.SemaphoreType.DMA,
            gs1=pltpu.SemaphoreType.DMA,
            gs2=pltpu.SemaphoreType.DMA,
            gs3=pltpu.SemaphoreType.DMA,
            ss0=pltpu.SemaphoreType.DMA,
            ss1=pltpu.SemaphoreType.DMA,
            ss2=pltpu.SemaphoreType.DMA,
            ss3=pltpu.SemaphoreType.DMA,
        ),
    )
    def run(hp_h, src_h, dst_h, ew_h, z_h, acc_h,
            *, srcb, dstb, ewb, rows, srct, dstt, ewt, rowst, acc,
            lsem, gs0, gs1, gs2, gs3, ss0, ss1, ss2, ss3):
        _agg_body(hp_h, src_h, dst_h, ew_h, z_h, acc_h,
                  srcb, dstb, ewb, rows, srct, dstt, ewt, rowst, acc,
                  lsem, [gs0, gs1, gs2, gs3], [ss0, ss1, ss2, ss3])

    return run(hp, src1d, dst1d, ew1d, zeros16)


FLAT = N_PAD * H // 128
_FB = 256
_GRID = FLAT // _FB


def _prologue_tc(deg_ref, x_ref, wbd_ref, dis_ref, hp_ref):
    deg = deg_ref[0] + deg_ref[1] + 1.0
    dis = lax.rsqrt(deg)
    dis_ref[...] = dis
    h = jnp.dot(x_ref[...], wbd_ref[...], preferred_element_type=F32)
    hp_ref[...] = h * dis


def _run_prologue(deg_f, x_f, wbd1):
    return pl.pallas_call(
        _prologue_tc,
        grid=(_GRID,),
        in_specs=[
            pl.BlockSpec((NC, _FB, 128), lambda i: (0, i, 0)),
            pl.BlockSpec((_FB, 128), lambda i: (i, 0)),
            pl.BlockSpec((128, 128), lambda i: (0, 0)),
        ],
        out_specs=[
            pl.BlockSpec((_FB, 128), lambda i: (i, 0)),
            pl.BlockSpec((_FB, 128), lambda i: (i, 0)),
        ],
        out_shape=[
            jax.ShapeDtypeStruct((FLAT, 128), F32),
            jax.ShapeDtypeStruct((FLAT, 128), F32),
        ],
    )(deg_f, x_f, wbd1)


def _layer_tc(acc_ref, hp_ref, dis_ref, b_ref, wbd_ref, out_ref, *, last):
    s = acc_ref[0] + acc_ref[1] + hp_ref[...]
    v = dis_ref[...] * s + b_ref[...]
    x = jnp.where(v >= 0, v, 0.01 * v)
    if last:
        out_ref[...] = x
    else:
        h = jnp.dot(x, wbd_ref[...], preferred_element_type=F32)
        out_ref[...] = h * dis_ref[...]


def _run_layer(acc_f, hp_f, dis_f, b_f, wbd_next, last):
    return pl.pallas_call(
        functools.partial(_layer_tc, last=last),
        grid=(_GRID,),
        in_specs=[
            pl.BlockSpec((NC, _FB, 128), lambda i: (0, i, 0)),
            pl.BlockSpec((_FB, 128), lambda i: (i, 0)),
            pl.BlockSpec((_FB, 128), lambda i: (i, 0)),
            pl.BlockSpec((1, 128), lambda i: (0, 0)),
            pl.BlockSpec((128, 128), lambda i: (0, 0)),
        ],
        out_specs=pl.BlockSpec((_FB, 128), lambda i: (i, 0)),
        out_shape=jax.ShapeDtypeStruct((FLAT, 128), F32),
    )(acc_f, hp_f, dis_f, b_f, wbd_next)


def _head_tc(xa_ref, xb_ref, w1_ref, b1_ref, w2_ref, b2_ref, w3_ref, b3_ref,
             za_ref, penal_ref):
    def mlp(xg):
        z = jnp.dot(xg, w1_ref[...], preferred_element_type=F32) + b1_ref[...]
        z = jnp.dot(z, w2_ref[...], preferred_element_type=F32) + b2_ref[...]
        z = jnp.dot(z, w3_ref[...], preferred_element_type=F32) + b3_ref[...]
        return z

    za = mlp(xa_ref[...])
    zb = mlp(xb_ref[...])
    za_ref[...] = za
    d = za - zb
    ss = jnp.sum(d * d)
    penal = 0.09 * xa_ref.shape[0] * lax.rsqrt(ss)
    penal_ref[...] = jnp.reshape(penal, (1, 1))


def _run_head(xa, xb, w1, b1, w2, b2, w3, b3):
    rows = xa.shape[0]
    return pl.pallas_call(
        _head_tc,
        out_shape=[
            jax.ShapeDtypeStruct((rows, 128), F32),
            jax.ShapeDtypeStruct((1, 1), F32),
        ],
    )(xa, xb, w1, b1, w2, b2, w3, b3)


def kernel(x, edge_index, edge_weight, W1, b1, W2, b2, W3, b3, W4, b4,
           fc1_w, fc1_b, fc2_w, fc2_b, fc3_w, fc3_b):
    src = edge_index[0]
    dst = edge_index[1]

    zeros1d = jnp.zeros((N_PAD,), F32)
    zeros16 = jnp.zeros((N_PAD, H), F32)
    eye8 = jnp.eye(8, dtype=F32)
    wbds = [jnp.kron(eye8, w) for w in (W1, W2, W3, W4)]
    bfs = [jnp.tile(b, 8).reshape(1, 128) for b in (b1, b2, b3, b4)]

    x_f = jnp.pad(x.reshape(N * H // 128, 128),
                  ((0, FLAT - N * H // 128), (0, 0)))

    deg16 = _run_deg(dst, edge_weight, zeros1d)
    deg_f = deg16.reshape(NC, FLAT, 128)
    dis_f, hpf = _run_prologue(deg_f, x_f, wbds[0])

    for l in range(4):
        accp = _run_agg(hpf.reshape(N_PAD, H), src, dst, edge_weight,
                        zeros16)
        hpf = _run_layer(accp.reshape(NC, FLAT, 128), hpf, dis_f, bfs[l],
                         wbds[min(l + 1, 3)], last=(l == 3))

    xa = hpf[:N * H // 128].reshape(-1, 1600)
    xb = x.reshape(-1, 1600)
    fc3_wp = jnp.pad(fc3_w, ((0, 0), (0, 126)))
    fc3_bp = jnp.pad(fc3_b, (0, 126))
    za, penal = _run_head(xa, xb,
                          fc1_w, fc1_b.reshape(1, -1),
                          fc2_w, fc2_b.reshape(1, -1),
                          fc3_wp, fc3_bp.reshape(1, -1))
    x_cls = za[:, :2]
    return (x_cls, penal[0, 0])
